# Initial kernel scaffold; baseline (speedup 1.0000x reference)
import jax
import jax.numpy as jnp
from jax.experimental import pallas as pl


def kernel(x, am, W, b, gamma, beta):
    raise NotImplementedError("write your pallas kernel here")


# speedup vs baseline: 1.0199x; 1.0199x over previous
import jax
import jax.numpy as jnp
from jax import lax
from jax.experimental import pallas as pl
from jax.experimental.pallas import tpu as pltpu

_N = 10000
_D = 128
_BM = 200
_MB = _N // _BM


def _fused_body(x_ref, w_ref, g_ref, be_ref, am_ref, out_ref, xw_ref, s1_ref, s2_ref):
    i = pl.program_id(0)

    @pl.when(i == 0)
    def _init():
        xw_ref[...] = lax.dot_general(
            x_ref[...], w_ref[...],
            dimension_numbers=(((1,), (1,)), ((), ())),
            precision=lax.Precision.HIGHEST,
            preferred_element_type=jnp.float32,
        )
        s1_ref[...] = jnp.zeros_like(s1_ref)
        s2_ref[...] = jnp.zeros_like(s2_ref)

    h = lax.dot_general(
        am_ref[...], xw_ref[...],
        dimension_numbers=(((1,), (0,)), ((), ())),
        precision=lax.Precision.DEFAULT,
        preferred_element_type=jnp.float32,
    )
    out_ref[pl.ds(i * _BM, _BM), :] = h
    s1_ref[...] += jnp.sum(h, axis=0, keepdims=True)
    s2_ref[...] += jnp.sum(h * h, axis=0, keepdims=True)

    @pl.when(i == _MB - 1)
    def _finalize():
        inv_n = jnp.float32(1.0 / _N)
        mean = s1_ref[...] * inv_n
        var = s2_ref[...] * inv_n - mean * mean
        scale = g_ref[...] * lax.rsqrt(var + 1e-5)
        shift = be_ref[...] - mean * scale
        out_ref[...] = jnp.maximum(out_ref[...] * scale + shift, 0.0)


def kernel(x, am, W, b, gamma, beta):
    del b
    g2 = gamma.reshape(1, _D)
    be2 = beta.reshape(1, _D)
    return pl.pallas_call(
        _fused_body,
        grid=(_MB,),
        in_specs=[
            pl.BlockSpec((_N, _D), lambda i: (0, 0)),
            pl.BlockSpec((_D, _D), lambda i: (0, 0)),
            pl.BlockSpec((1, _D), lambda i: (0, 0)),
            pl.BlockSpec((1, _D), lambda i: (0, 0)),
            pl.BlockSpec((_BM, _N), lambda i: (i, 0)),
        ],
        out_specs=pl.BlockSpec((_N, _D), lambda i: (0, 0)),
        out_shape=jax.ShapeDtypeStruct((_N, _D), jnp.float32),
        scratch_shapes=[
            pltpu.VMEM((_N, _D), jnp.float32),
            pltpu.VMEM((1, _D), jnp.float32),
            pltpu.VMEM((1, _D), jnp.float32),
        ],
    )(x, W, g2, be2, am)
